# baseline (speedup 1.0000x reference)
import functools

import jax
import jax.numpy as jnp
from jax import lax
from jax.experimental import pallas as pl
from jax.experimental.pallas import tpu as pltpu
from jax.experimental.pallas import tpu_sc as plsc

N = 10000
E = 320000
D = 128
D_OUT = 64

NC = 2
NS = 16
NW = NC * NS
E_PER_W = E // NW
CH = 80
NCHUNK = E_PER_W // CH
ROWS_PER_S = 640
N_PAD = NS * ROWS_PER_S

_mesh = plsc.VectorSubcoreMesh(core_axis_name="c", subcore_axis_name="s")


NBUF = 4
IBUF = 2 * NBUF
NROUND = (NCHUNK + IBUF - 1) // IBUF

PCH = CH
PNCHUNK = E_PER_W // PCH
PRING = 4
PIBUF = 8
PLOOK = 2
PNROUND = (PNCHUNK + PLOOK + PIBUF - 1) // PIBUF


@functools.partial(
    pl.kernel,
    out_type=jax.ShapeDtypeStruct((NC, N_PAD, D), jnp.float32),
    mesh=_mesh,
    scratch_types=[
        pltpu.VMEM((PIBUF, PCH), jnp.int32),
        pltpu.VMEM((PIBUF, PCH), jnp.int32),
        pltpu.VMEM((PRING, PCH, D), jnp.float32),
        pltpu.VMEM_SHARED((N_PAD, D), jnp.float32),
    ] + [pltpu.SemaphoreType.DMA] * (PIBUF + 2 * PRING),
)
def _prop_kernel(g_hbm, src_hbm, dst_hbm, zeros_hbm, out,
                 srcb, dstb, rows, acc, *sems):
    isem = sems[:PIBUF]
    gsem = sems[PIBUF:PIBUF + PRING]
    ssem = sems[PIBUF + PRING:]
    c = lax.axis_index("c")
    s = lax.axis_index("s")
    wid = s * NC + c
    r0 = s * ROWS_PER_S

    def load_pair(i, sl):
        pltpu.async_copy(src_hbm.at[wid, i], srcb.at[sl], isem[sl])
        pltpu.async_copy(dst_hbm.at[wid, i], dstb.at[sl], isem[sl])

    def wait_pair(sl):
        pltpu.make_async_copy(src_hbm.at[wid, 0], srcb.at[sl],
                              isem[sl]).wait()
        pltpu.make_async_copy(dst_hbm.at[wid, 0], dstb.at[sl],
                              isem[sl]).wait()

    def start_gather(sl, b):
        pltpu.async_copy(g_hbm.at[srcb.at[sl]], rows.at[b], gsem[b])

    def wait_gather(b):
        pltpu.make_async_copy(g_hbm.at[srcb.at[0]], rows.at[b],
                              gsem[b]).wait()

    def start_scatter(sl, b):
        pltpu.async_copy(rows.at[b], acc.at[dstb.at[sl]], ssem[b], add=True)

    def wait_scatter(b):
        pltpu.make_async_copy(rows.at[b], acc.at[dstb.at[0]], ssem[b]).wait()

    for j in range(PIBUF):
        load_pair(j, j)
    pltpu.sync_copy(zeros_hbm.at[pl.ds(r0, ROWS_PER_S), :],
                    acc.at[pl.ds(r0, ROWS_PER_S), :])
    plsc.subcore_barrier()
    for b in range(PLOOK):
        wait_pair(b)
        start_gather(b, b)

    def outer(g, carry):
        for j in range(PIBUF):
            b = j % PRING
            i = g * PIBUF + j

            @pl.when(i < PNCHUNK)
            def _():
                wait_gather(b)
                start_scatter(j, b)

            @pl.when((i >= PLOOK) & (i - PLOOK < PNCHUNK))
            def _():
                wait_scatter((j + PRING - PLOOK) % PRING)

            @pl.when((i >= PLOOK) & (i + PIBUF - PLOOK < PNCHUNK))
            def _():
                load_pair(i + PIBUF - PLOOK, (j + PIBUF - PLOOK) % PIBUF)

            @pl.when(i + PLOOK < PNCHUNK)
            def _():
                wait_pair((j + PLOOK) % PIBUF)
                start_gather((j + PLOOK) % PIBUF, (j + PLOOK) % PRING)

        return carry

    lax.fori_loop(0, PNROUND, outer, 0)
    plsc.subcore_barrier()
    pltpu.sync_copy(acc.at[pl.ds(r0, ROWS_PER_S), :],
                    out.at[c, pl.ds(r0, ROWS_PER_S), :])


@functools.partial(
    pl.kernel,
    out_type=jax.ShapeDtypeStruct((NC, N_PAD, D), jnp.float32),
    mesh=_mesh,
    scratch_types=[
        pltpu.VMEM((IBUF, CH), jnp.int32),
        pltpu.VMEM((CH, D), jnp.float32),
        pltpu.VMEM_SHARED((N_PAD, D), jnp.float32),
    ] + [pltpu.SemaphoreType.DMA] * IBUF,
)
def _deg_kernel(dst_hbm, ones_hbm, zeros_hbm, out, dstb, onesb, acc, *isem):
    c = lax.axis_index("c")
    s = lax.axis_index("s")
    wid = s * NC + c
    r0 = s * ROWS_PER_S

    def load_idx(i, sl):
        pltpu.async_copy(dst_hbm.at[wid, i], dstb.at[sl], isem[sl])

    def wait_idx(sl):
        pltpu.make_async_copy(dst_hbm.at[wid, 0], dstb.at[sl], isem[sl]).wait()

    for j in range(IBUF):
        load_idx(j, j)
    pltpu.sync_copy(ones_hbm, onesb)
    pltpu.sync_copy(zeros_hbm.at[pl.ds(r0, ROWS_PER_S), :],
                    acc.at[pl.ds(r0, ROWS_PER_S), :])
    plsc.subcore_barrier()

    def outer(g, carry):
        for j in range(IBUF):
            i = g * IBUF + j

            @pl.when(i < NCHUNK)
            def _():
                wait_idx(j)
                pltpu.sync_copy(onesb, acc.at[dstb.at[j]], add=True)

            @pl.when(i + IBUF < NCHUNK)
            def _():
                load_idx(i + IBUF, j)

        return carry

    lax.fori_loop(0, NROUND, outer, 0)
    plsc.subcore_barrier()
    pltpu.sync_copy(acc.at[pl.ds(r0, ROWS_PER_S), :],
                    out.at[c, pl.ds(r0, ROWS_PER_S), :])


_BLK = 1000
_GRID = N // _BLK


def _dinv_block(d0, d1):
    deg = d0[0, :, 0:1] + d1[0, :, 0:1]
    return lax.rsqrt(jnp.maximum(deg, 1.0))


def _tc_in_body(x_ref, w_ref, b_ref, d0_ref, d1_ref, h_ref, g_ref):
    h = jnp.dot(x_ref[...], w_ref[...], preferred_element_type=jnp.float32)
    h = h + b_ref[...]
    h_ref[...] = h
    g_ref[...] = h * _dinv_block(d0_ref[...], d1_ref[...])


def _tc_mid_body(p0_ref, p1_ref, d0_ref, d1_ref, g_ref):
    deg = d0_ref[0, :, 0:1] + d1_ref[0, :, 0:1]
    g_ref[...] = (p0_ref[0] + p1_ref[0]) / jnp.maximum(deg, 1.0)


def _tc_block_end_body(p0_ref, p1_ref, d0_ref, d1_ref, xres_ref, w_ref, b_ref,
                       x1_ref, g_ref):
    dinv = _dinv_block(d0_ref[...], d1_ref[...])
    t = (p0_ref[0] + p1_ref[0]) * dinv
    x1 = jnp.dot(t, w_ref[...], preferred_element_type=jnp.float32)
    x1 = x1 + b_ref[...] + xres_ref[...]
    x1_ref[...] = x1
    g_ref[...] = x1 * dinv


def _tc_final_body(p0_ref, p1_ref, d0_ref, d1_ref, xres_ref, w_ref, b_ref,
                   wo_ref, bo_ref, out_ref):
    dinv = _dinv_block(d0_ref[...], d1_ref[...])
    t = (p0_ref[0] + p1_ref[0]) * dinv
    x2 = jnp.dot(t, w_ref[...], preferred_element_type=jnp.float32)
    x2 = x2 + b_ref[...] + xres_ref[...]
    o = jnp.dot(x2, wo_ref[...], preferred_element_type=jnp.float32)
    o = o + bo_ref[...]
    m = jnp.max(o, axis=1, keepdims=True)
    ex = jnp.exp(o - m)
    out_ref[...] = o - m - jnp.log(jnp.sum(ex, axis=1, keepdims=True))


def _rows_spec(w):
    return pl.BlockSpec((_BLK, w), lambda i: (i, 0))


def _plane_spec(w, plane):
    return pl.BlockSpec((1, _BLK, w), lambda i, _p=plane: (_p, i, 0))


def _full_spec(a, b):
    return pl.BlockSpec((a, b), lambda i: (0, 0))


_f32 = jnp.float32


def _tc_in(x, w, b, dparts):
    return pl.pallas_call(
        _tc_in_body,
        grid=(_GRID,),
        in_specs=[_rows_spec(D), _full_spec(D, D), _full_spec(1, D),
                  _plane_spec(D, 0), _plane_spec(D, 1)],
        out_specs=[_rows_spec(D), _rows_spec(D)],
        out_shape=[jax.ShapeDtypeStruct((N, D), _f32)] * 2,
    )(x, w, b, dparts, dparts)


def _tc_mid(p, dparts):
    return pl.pallas_call(
        _tc_mid_body,
        grid=(_GRID,),
        in_specs=[_plane_spec(D, 0), _plane_spec(D, 1),
                  _plane_spec(D, 0), _plane_spec(D, 1)],
        out_specs=[_rows_spec(D)],
        out_shape=[jax.ShapeDtypeStruct((N, D), _f32)],
    )(p, p, dparts, dparts)[0]


def _tc_block_end(p, dparts, xres, w, b):
    return pl.pallas_call(
        _tc_block_end_body,
        grid=(_GRID,),
        in_specs=[_plane_spec(D, 0), _plane_spec(D, 1),
                  _plane_spec(D, 0), _plane_spec(D, 1),
                  _rows_spec(D), _full_spec(D, D), _full_spec(1, D)],
        out_specs=[_rows_spec(D), _rows_spec(D)],
        out_shape=[jax.ShapeDtypeStruct((N, D), _f32)] * 2,
    )(p, p, dparts, dparts, xres, w, b)


def _tc_final(p, dparts, xres, w, b, wo, bo):
    return pl.pallas_call(
        _tc_final_body,
        grid=(_GRID,),
        in_specs=[_plane_spec(D, 0), _plane_spec(D, 1),
                  _plane_spec(D, 0), _plane_spec(D, 1),
                  _rows_spec(D), _full_spec(D, D), _full_spec(1, D),
                  _full_spec(D, D_OUT), _full_spec(1, D_OUT)],
        out_specs=[_rows_spec(D_OUT)],
        out_shape=[jax.ShapeDtypeStruct((N, D_OUT), _f32)],
    )(p, p, dparts, dparts, xres, w, b, wo, bo)[0]


def kernel(x, adj, W_in, b_in, W_mid1, b_mid1, W_mid2, b_mid2, W_out, b_out):
    src = adj[0].reshape(NW, PNCHUNK, PCH)
    dst3 = adj[1].reshape(NW, PNCHUNK, PCH)
    dstd = adj[1].reshape(NW, NCHUNK, CH)
    zeros_nd = jnp.zeros((N_PAD, D), jnp.float32)
    ones_ch = jnp.ones((CH, D), jnp.float32)

    dparts = _deg_kernel(dstd, ones_ch, zeros_nd)

    h, g0 = _tc_in(x, W_in, b_in.reshape(1, D), dparts)

    p = _prop_kernel(g0, src, dst3, zeros_nd)
    g1 = _tc_mid(p, dparts)
    p = _prop_kernel(g1, src, dst3, zeros_nd)
    x1, g2 = _tc_block_end(p, dparts, h, W_mid1, b_mid1.reshape(1, D))

    p = _prop_kernel(g2, src, dst3, zeros_nd)
    g3 = _tc_mid(p, dparts)
    p = _prop_kernel(g3, src, dst3, zeros_nd)
    out = _tc_final(p, dparts, x1, W_mid2, b_mid2.reshape(1, D),
                    W_out, b_out.reshape(1, D_OUT))
    return out


# speedup vs baseline: 1.1621x; 1.1621x over previous
import functools

import jax
import jax.numpy as jnp
from jax import lax
from jax.experimental import pallas as pl
from jax.experimental.pallas import tpu as pltpu
from jax.experimental.pallas import tpu_sc as plsc

N = 10000
E = 320000
D = 128
D_OUT = 64

NC = 2
NS = 16
NW = NC * NS
E_PER_W = E // NW
CH = 80
NCHUNK = E_PER_W // CH
ROWS_PER_S = 640
N_PAD = NS * ROWS_PER_S

_mesh = plsc.VectorSubcoreMesh(core_axis_name="c", subcore_axis_name="s")


NBUF = 4
IBUF = 2 * NBUF
NROUND = (NCHUNK + IBUF - 1) // IBUF


@functools.partial(
    pl.kernel,
    out_type=jax.ShapeDtypeStruct((NC, N_PAD, D), jnp.float32),
    mesh=_mesh,
    scratch_types=[
        pltpu.VMEM((IBUF, CH), jnp.int32),
        pltpu.VMEM((IBUF, CH), jnp.int32),
        pltpu.VMEM((NBUF, CH, D), jnp.float32),
        pltpu.VMEM_SHARED((N_PAD, D), jnp.float32),
    ] + [pltpu.SemaphoreType.DMA] * (NBUF + IBUF),
)
def _prop_kernel(g_hbm, src_hbm, dst_hbm, zeros_hbm, out,
                 srcb, dstb, rows, acc, *sems):
    gsem = sems[:NBUF]
    isem = sems[NBUF:]
    c = lax.axis_index("c")
    s = lax.axis_index("s")
    wid = s * NC + c
    r0 = s * ROWS_PER_S

    def load_idx(i, sl):
        pltpu.async_copy(src_hbm.at[wid, i], srcb.at[sl], isem[sl])
        pltpu.async_copy(dst_hbm.at[wid, i], dstb.at[sl], isem[sl])

    def wait_idx(sl):
        pltpu.make_async_copy(src_hbm.at[wid, 0], srcb.at[sl], isem[sl]).wait()
        pltpu.make_async_copy(dst_hbm.at[wid, 0], dstb.at[sl], isem[sl]).wait()

    def start_gather(sl, b):
        pltpu.async_copy(g_hbm.at[srcb.at[sl]], rows.at[b], gsem[b])

    def wait_gather(b):
        pltpu.make_async_copy(g_hbm.at[srcb.at[0]], rows.at[b],
                              gsem[b]).wait()

    for j in range(IBUF):
        load_idx(j, j)
    pltpu.sync_copy(zeros_hbm.at[pl.ds(r0, ROWS_PER_S), :],
                    acc.at[pl.ds(r0, ROWS_PER_S), :])
    plsc.subcore_barrier()
    for b in range(NBUF):
        wait_idx(b)
        start_gather(b, b)

    def outer(g, carry):
        for j in range(IBUF):
            b = j % NBUF
            i = g * IBUF + j

            @pl.when(i < NCHUNK)
            def _():
                wait_gather(b)
                pltpu.sync_copy(rows.at[b], acc.at[dstb.at[j]], add=True)

            @pl.when(i + IBUF < NCHUNK)
            def _():
                load_idx(i + IBUF, j)

            @pl.when(i + NBUF < NCHUNK)
            def _():
                wait_idx((j + NBUF) % IBUF)
                start_gather((j + NBUF) % IBUF, b)

        return carry

    lax.fori_loop(0, NROUND, outer, 0)
    plsc.subcore_barrier()
    pltpu.sync_copy(acc.at[pl.ds(r0, ROWS_PER_S), :],
                    out.at[c, pl.ds(r0, ROWS_PER_S), :])


@functools.partial(
    pl.kernel,
    out_type=jax.ShapeDtypeStruct((NC, N_PAD, D), jnp.float32),
    mesh=_mesh,
    scratch_types=[
        pltpu.VMEM((IBUF, CH), jnp.int32),
        pltpu.VMEM((CH, D), jnp.float32),
        pltpu.VMEM_SHARED((N_PAD, D), jnp.float32),
    ] + [pltpu.SemaphoreType.DMA] * IBUF,
)
def _deg_kernel(dst_hbm, ones_hbm, zeros_hbm, out, dstb, onesb, acc, *isem):
    c = lax.axis_index("c")
    s = lax.axis_index("s")
    wid = s * NC + c
    r0 = s * ROWS_PER_S

    def load_idx(i, sl):
        pltpu.async_copy(dst_hbm.at[wid, i], dstb.at[sl], isem[sl])

    def wait_idx(sl):
        pltpu.make_async_copy(dst_hbm.at[wid, 0], dstb.at[sl], isem[sl]).wait()

    for j in range(IBUF):
        load_idx(j, j)
    pltpu.sync_copy(ones_hbm, onesb)
    pltpu.sync_copy(zeros_hbm.at[pl.ds(r0, ROWS_PER_S), :],
                    acc.at[pl.ds(r0, ROWS_PER_S), :])
    plsc.subcore_barrier()

    def outer(g, carry):
        for j in range(IBUF):
            i = g * IBUF + j

            @pl.when(i < NCHUNK)
            def _():
                wait_idx(j)
                pltpu.sync_copy(onesb, acc.at[dstb.at[j]], add=True)

            @pl.when(i + IBUF < NCHUNK)
            def _():
                load_idx(i + IBUF, j)

        return carry

    lax.fori_loop(0, NROUND, outer, 0)
    plsc.subcore_barrier()
    pltpu.sync_copy(acc.at[pl.ds(r0, ROWS_PER_S), :],
                    out.at[c, pl.ds(r0, ROWS_PER_S), :])


_BLK = 1000
_GRID = N // _BLK


def _dinv_block(d0, d1):
    deg = d0[0, :, 0:1] + d1[0, :, 0:1]
    return lax.rsqrt(jnp.maximum(deg, 1.0))


def _tc_in_body(x_ref, w_ref, b_ref, d0_ref, d1_ref, h_ref, g_ref):
    h = jnp.dot(x_ref[...], w_ref[...], preferred_element_type=jnp.float32)
    h = h + b_ref[...]
    h_ref[...] = h
    g_ref[...] = h * _dinv_block(d0_ref[...], d1_ref[...])


def _tc_mid_body(p0_ref, p1_ref, d0_ref, d1_ref, g_ref):
    deg = d0_ref[0, :, 0:1] + d1_ref[0, :, 0:1]
    g_ref[...] = (p0_ref[0] + p1_ref[0]) / jnp.maximum(deg, 1.0)


def _tc_block_end_body(p0_ref, p1_ref, d0_ref, d1_ref, xres_ref, w_ref, b_ref,
                       x1_ref, g_ref):
    dinv = _dinv_block(d0_ref[...], d1_ref[...])
    t = (p0_ref[0] + p1_ref[0]) * dinv
    x1 = jnp.dot(t, w_ref[...], preferred_element_type=jnp.float32)
    x1 = x1 + b_ref[...] + xres_ref[...]
    x1_ref[...] = x1
    g_ref[...] = x1 * dinv


def _tc_final_body(p0_ref, p1_ref, d0_ref, d1_ref, xres_ref, w_ref, b_ref,
                   wo_ref, bo_ref, out_ref):
    dinv = _dinv_block(d0_ref[...], d1_ref[...])
    t = (p0_ref[0] + p1_ref[0]) * dinv
    x2 = jnp.dot(t, w_ref[...], preferred_element_type=jnp.float32)
    x2 = x2 + b_ref[...] + xres_ref[...]
    o = jnp.dot(x2, wo_ref[...], preferred_element_type=jnp.float32)
    o = o + bo_ref[...]
    m = jnp.max(o, axis=1, keepdims=True)
    ex = jnp.exp(o - m)
    out_ref[...] = o - m - jnp.log(jnp.sum(ex, axis=1, keepdims=True))


def _rows_spec(w):
    return pl.BlockSpec((_BLK, w), lambda i: (i, 0))


def _plane_spec(w, plane):
    return pl.BlockSpec((1, _BLK, w), lambda i, _p=plane: (_p, i, 0))


def _full_spec(a, b):
    return pl.BlockSpec((a, b), lambda i: (0, 0))


_f32 = jnp.float32


def _tc_in(x, w, b, dparts):
    return pl.pallas_call(
        _tc_in_body,
        grid=(_GRID,),
        in_specs=[_rows_spec(D), _full_spec(D, D), _full_spec(1, D),
                  _plane_spec(D, 0), _plane_spec(D, 1)],
        out_specs=[_rows_spec(D), _rows_spec(D)],
        out_shape=[jax.ShapeDtypeStruct((N, D), _f32)] * 2,
    )(x, w, b, dparts, dparts)


def _tc_mid(p, dparts):
    return pl.pallas_call(
        _tc_mid_body,
        grid=(_GRID,),
        in_specs=[_plane_spec(D, 0), _plane_spec(D, 1),
                  _plane_spec(D, 0), _plane_spec(D, 1)],
        out_specs=[_rows_spec(D)],
        out_shape=[jax.ShapeDtypeStruct((N, D), _f32)],
    )(p, p, dparts, dparts)[0]


def _tc_block_end(p, dparts, xres, w, b):
    return pl.pallas_call(
        _tc_block_end_body,
        grid=(_GRID,),
        in_specs=[_plane_spec(D, 0), _plane_spec(D, 1),
                  _plane_spec(D, 0), _plane_spec(D, 1),
                  _rows_spec(D), _full_spec(D, D), _full_spec(1, D)],
        out_specs=[_rows_spec(D), _rows_spec(D)],
        out_shape=[jax.ShapeDtypeStruct((N, D), _f32)] * 2,
    )(p, p, dparts, dparts, xres, w, b)


def _tc_final(p, dparts, xres, w, b, wo, bo):
    return pl.pallas_call(
        _tc_final_body,
        grid=(_GRID,),
        in_specs=[_plane_spec(D, 0), _plane_spec(D, 1),
                  _plane_spec(D, 0), _plane_spec(D, 1),
                  _rows_spec(D), _full_spec(D, D), _full_spec(1, D),
                  _full_spec(D, D_OUT), _full_spec(1, D_OUT)],
        out_specs=[_rows_spec(D_OUT)],
        out_shape=[jax.ShapeDtypeStruct((N, D_OUT), _f32)],
    )(p, p, dparts, dparts, xres, w, b, wo, bo)[0]


def kernel(x, adj, W_in, b_in, W_mid1, b_mid1, W_mid2, b_mid2, W_out, b_out):
    src = adj[0].reshape(NW, NCHUNK, CH)
    dst3 = adj[1].reshape(NW, NCHUNK, CH)
    zeros_nd = jnp.zeros((N_PAD, D), jnp.float32)
    ones_ch = jnp.ones((CH, D), jnp.float32)

    dparts = _deg_kernel(dst3, ones_ch, zeros_nd)

    h, g0 = _tc_in(x, W_in, b_in.reshape(1, D), dparts)

    p = _prop_kernel(g0, src, dst3, zeros_nd)
    g1 = _tc_mid(p, dparts)
    p = _prop_kernel(g1, src, dst3, zeros_nd)
    x1, g2 = _tc_block_end(p, dparts, h, W_mid1, b_mid1.reshape(1, D))

    p = _prop_kernel(g2, src, dst3, zeros_nd)
    g3 = _tc_mid(p, dparts)
    p = _prop_kernel(g3, src, dst3, zeros_nd)
    out = _tc_final(p, dparts, x1, W_mid2, b_mid2.reshape(1, D),
                    W_out, b_out.reshape(1, D_OUT))
    return out
